# Initial kernel scaffold; baseline (speedup 1.0000x reference)
import jax
import jax.numpy as jnp
from jax.experimental import pallas as pl


def kernel(input_bert_features, attention_mask, spans, span_mask, related_spans_tensor, sentence_length, Wi_f, bi_f, Wo_f, bo_f, g_f, be_f, Wi_r, bi_r, Wo_r, bo_r, g_r, be_r, Wa, ba, Wop, bop):
    raise NotImplementedError("write your pallas kernel here")


# speedup vs baseline: 7.3730x; 7.3730x over previous
import functools

import jax
import jax.numpy as jnp
from jax import lax
from jax.experimental import pallas as pl
from jax.experimental.pallas import tpu as pltpu

B, S, D = 8, 512, 768
SPAN_NUM = 2048
D_FF = 3072
N_CLS = 3
BN = 256
G = (B * SPAN_NUM) // BN
NB_PER_BATCH = SPAN_NUM // BN
_SQRT_HALF = 0.7071067811865476


def _ffn_ln(emb, wi_ref, bi_ref, wo_ref, bo_ref, g_ref, be_ref):
    inter = jnp.dot(emb, wi_ref[...], preferred_element_type=jnp.float32)
    inter = inter + bi_ref[...]
    inter = 0.5 * inter * (1.0 + lax.erf(inter * _SQRT_HALF))
    out = jnp.dot(inter, wo_ref[...], preferred_element_type=jnp.float32)
    out = out + bo_ref[...] + emb
    m = jnp.mean(out, axis=-1, keepdims=True)
    d = out - m
    v = jnp.mean(d * d, axis=-1, keepdims=True)
    return d * lax.rsqrt(v + 1e-12) * g_ref[...] + be_ref[...]


def _body(s_ref, e_ref, iw_ref, x_ref,
          wi_f_ref, bi_f_ref, wo_f_ref, bo_f_ref, g_f_ref, be_f_ref,
          wi_r_ref, bi_r_ref, wo_r_ref, bo_r_ref, g_r_ref, be_r_ref,
          wa_ref, ba_ref, wop_ref, bop_ref, out_ref):
    s = s_ref[0]
    e = e_ref[0]
    iw = iw_ref[0]
    t = lax.broadcasted_iota(jnp.int32, (BN, S), 1).astype(jnp.float32)
    a = jnp.where((t >= s) & (t <= e), iw, 0.0)
    emb = jnp.dot(a, x_ref[0], preferred_element_type=jnp.float32)

    y1 = _ffn_ln(emb, wi_f_ref, bi_f_ref, wo_f_ref, bo_f_ref, g_f_ref, be_f_ref)
    y2 = _ffn_ln(emb, wi_r_ref, bi_r_ref, wo_r_ref, bo_r_ref, g_r_ref, be_r_ref)
    c1 = jnp.dot(y1, wa_ref[...], preferred_element_type=jnp.float32) + ba_ref[...]
    c2 = jnp.dot(y2, wop_ref[...], preferred_element_type=jnp.float32) + bop_ref[...]
    out_ref[0] = jnp.concatenate([c1, c2], axis=-1)


@jax.jit
def _run(startf, endf, invw, x,
         Wi_f, bi_f, Wo_f, bo_f, g_f, be_f,
         Wi_r, bi_r, Wo_r, bo_r, g_r, be_r, Wa, ba, Wop, bop):
    const2 = pl.BlockSpec((1, D_FF), lambda i: (0, 0))
    constd = pl.BlockSpec((1, D), lambda i: (0, 0))
    w_big = pl.BlockSpec((D, D_FF), lambda i: (0, 0))
    w_big_t = pl.BlockSpec((D_FF, D), lambda i: (0, 0))
    w_cls = pl.BlockSpec((D, N_CLS), lambda i: (0, 0))
    b_cls = pl.BlockSpec((1, N_CLS), lambda i: (0, 0))
    span_spec = pl.BlockSpec((1, BN, 1), lambda i: (i, 0, 0))
    out = pl.pallas_call(
        _body,
        grid=(G,),
        in_specs=[
            span_spec, span_spec, span_spec,
            pl.BlockSpec((1, S, D), lambda i: (i // NB_PER_BATCH, 0, 0)),
            w_big, const2, w_big_t, constd, constd, constd,
            w_big, const2, w_big_t, constd, constd, constd,
            w_cls, b_cls, w_cls, b_cls,
        ],
        out_specs=pl.BlockSpec((1, BN, 2 * N_CLS), lambda i: (i, 0, 0)),
        out_shape=jax.ShapeDtypeStruct((G, BN, 2 * N_CLS), jnp.float32),
        compiler_params=pltpu.CompilerParams(
            dimension_semantics=("arbitrary",),
            vmem_limit_bytes=120 * 1024 * 1024,
        ),
    )(startf, endf, invw, x,
      Wi_f, bi_f.reshape(1, D_FF), Wo_f, bo_f.reshape(1, D),
      g_f.reshape(1, D), be_f.reshape(1, D),
      Wi_r, bi_r.reshape(1, D_FF), Wo_r, bo_r.reshape(1, D),
      g_r.reshape(1, D), be_r.reshape(1, D),
      Wa, ba.reshape(1, N_CLS), Wop, bop.reshape(1, N_CLS))
    return out.reshape(B, SPAN_NUM, 2 * N_CLS)


def kernel(input_bert_features, attention_mask, spans, span_mask,
           related_spans_tensor, sentence_length,
           Wi_f, bi_f, Wo_f, bo_f, g_f, be_f,
           Wi_r, bi_r, Wo_r, bo_r, g_r, be_r, Wa, ba, Wop, bop):
    start = spans[..., 0].astype(jnp.float32)
    width = spans[..., 2].astype(jnp.float32)
    end = start + width - 1.0
    invw = 1.0 / jnp.maximum(width, 1.0)
    mask = span_mask.astype(jnp.float32)
    invw = invw * mask
    startf = start.reshape(G, BN, 1)
    endf = end.reshape(G, BN, 1)
    invwf = invw.reshape(G, BN, 1)
    return _run(startf, endf, invwf, input_bert_features,
                Wi_f, bi_f, Wo_f, bo_f, g_f, be_f,
                Wi_r, bi_r, Wo_r, bo_r, g_r, be_r, Wa, ba, Wop, bop)
